# baseline (speedup 1.0000x reference)
import functools

import jax
import jax.numpy as jnp
from jax.experimental import pallas as pl
from jax.experimental.pallas import tpu as pltpu

_PAD = 256


def _block_kernel(xall, dw1t, w1f, b1, dw2t, w2f, b2, wr, o_ref, y1pad,
                  *, cin, ow, m, ch):
    nch = m // ch
    lane = jax.lax.broadcasted_iota(jnp.int32, (1, ch), 1)
    col = jnp.bitwise_and(lane, ow - 1)
    mask_l = jnp.where(col == 0, 0.0, 1.0)
    mask_r = jnp.where(col == ow - 1, 0.0, 1.0)

    def masked(t, dw):
        if dw == -1:
            return t * mask_l
        if dw == 1:
            return t * mask_r
        return t

    def tap_x(idx, dh, dw, c0):
        s = c0 + ow * dh + dw
        if s >= 0:
            t = xall[idx, 0, :, s:s + ch]
        else:
            t = jnp.concatenate(
                [jnp.zeros((cin, -s), jnp.float32), xall[idx, 0, :, 0:ch + s]],
                axis=-1)
        return masked(t, dw)

    def tap_y(dh, dw, c0):
        s = _PAD + c0 + ow * dh + dw
        return masked(y1pad[:, s:s + ch], dw)

    def tap1(k):
        return (0, 0) if k == 1 else ((1, -1) if k == 0 else (1, 0))

    y1pad[:, :_PAD] = jnp.zeros_like(y1pad[:, :_PAD])
    y1pad[:, _PAD + m:] = jnp.zeros_like(y1pad[:, _PAD + m:])
    for ci in range(nch):
        c0 = ci * ch
        d1 = None
        for kh in range(3):
            a, dh = tap1(kh)
            for kw in range(3):
                b, dw = tap1(kw)
                c = dw1t[:, kh * 3 + kw][:, None] * tap_x(2 * a + b, dh, dw, c0)
                d1 = c if d1 is None else d1 + c
        y1 = jnp.dot(w1f[...], d1, preferred_element_type=jnp.float32) + b1[...]
        y1pad[:, _PAD + c0:_PAD + c0 + ch] = jnp.maximum(y1, 0.0)

    for ci in range(nch):
        c0 = ci * ch
        d2 = None
        for kh in range(3):
            for kw in range(3):
                c = dw2t[:, kh * 3 + kw][:, None] * tap_y(kh - 1, kw - 1, c0)
                d2 = c if d2 is None else d2 + c
        y2 = jnp.dot(w2f[...], d2, preferred_element_type=jnp.float32) + b2[...]
        res = jnp.dot(wr[...], xall[0, 0, :, c0:c0 + ch],
                      preferred_element_type=jnp.float32)
        o_ref[0, :, c0:c0 + ch] = jnp.maximum(y2 + res, 0.0)


def kernel(dw1, pw1, s1, b1, dw2, pw2, s2, b2, w1x1, x):
    n, cin, h, w = x.shape
    oh, ow = h // 2, w // 2
    hw = oh * ow
    m = hw
    cout = pw1.shape[1]

    xr = x.reshape(n, cin, oh, 2, ow, 2)
    xall = xr.transpose(3, 5, 0, 1, 2, 4)
    xall = xall.reshape(4, n, cin, m)

    w1f = pw1.T * s1[:, None]
    w2f = pw2.T * s2[:, None]
    wr = w1x1.T
    b1r = b1[:, None]
    b2r = b2[:, None]
    dw1t = dw1.T
    dw2t = dw2.T

    ch = min(hw, 4096)
    body = functools.partial(_block_kernel, cin=cin, ow=ow, m=m, ch=ch)

    resident = lambda s: pl.BlockSpec(s, lambda i: (0, 0))

    out = pl.pallas_call(
        body,
        out_shape=jax.ShapeDtypeStruct((n, cout, hw), jnp.float32),
        grid=(n,),
        in_specs=[pl.BlockSpec((4, 1, cin, m), lambda i: (0, i, 0, 0))] + [
            resident(dw1t.shape), resident(w1f.shape), resident(b1r.shape),
            resident(dw2t.shape), resident(w2f.shape), resident(b2r.shape),
            resident(wr.shape),
        ],
        out_specs=pl.BlockSpec((1, cout, hw), lambda i: (i, 0, 0)),
        scratch_shapes=[pltpu.VMEM((cout, _PAD + m + 256), jnp.float32)],
        compiler_params=pltpu.CompilerParams(
            dimension_semantics=("parallel",),
            vmem_limit_bytes=50 * 1024 * 1024,
        ),
    )(xall, dw1t, w1f, b1r, dw2t, w2f, b2r, wr)

    return out.reshape(n, cout, oh, ow)


# speedup vs baseline: 1.2687x; 1.2687x over previous
import functools

import jax
import jax.numpy as jnp
from jax.experimental import pallas as pl
from jax.experimental.pallas import tpu as pltpu

_PAD = 256


def _block_kernel(p00, p01, p10, p11, w1f, b1, w2f, b2, wr,
                  o_ref, y1pad, pscr, *, cin, ow, m, ch):
    for i, ref in enumerate((p00, p01, p10, p11)):
        pscr[i] = ref[0].astype(jnp.float32)
    phases = {(0, 0): 0, (0, 1): 1, (1, 0): 2, (1, 1): 3}
    nch = m // ch
    lane = jax.lax.broadcasted_iota(jnp.int32, (1, ch), 1)
    col = jnp.bitwise_and(lane, ow - 1)
    mask_l = jnp.where(col == 0, 0.0, 1.0)
    mask_r = jnp.where(col == ow - 1, 0.0, 1.0)

    def masked(t, dw):
        if dw == -1:
            return t * mask_l
        if dw == 1:
            return t * mask_r
        return t

    def tap_x(idx, dh, dw, c0):
        i = phases[idx]
        s = c0 + ow * dh + dw
        if s >= 0:
            t = pscr[i, :, s:s + ch]
        else:
            t = jnp.concatenate(
                [jnp.zeros((cin, -s), jnp.float32), pscr[i, :, 0:ch + s]],
                axis=-1)
        return masked(t, dw)

    def tap_y(dh, dw, c0):
        s = _PAD + c0 + ow * dh + dw
        return masked(y1pad[:, s:s + ch], dw)

    def tap1(k):
        return (0, 0) if k == 1 else ((1, -1) if k == 0 else (1, 0))

    y1pad[:, :_PAD] = jnp.zeros_like(y1pad[:, :_PAD])
    y1pad[:, _PAD + m:] = jnp.zeros_like(y1pad[:, _PAD + m:])
    for ci in range(nch):
        c0 = ci * ch
        taps = []
        for kh in range(3):
            a, dh = tap1(kh)
            for kw in range(3):
                b, dw = tap1(kw)
                taps.append(tap_x((a, b), dh, dw, c0))
        d1 = jnp.concatenate(taps, axis=0)
        y1 = jnp.dot(w1f[...], d1, preferred_element_type=jnp.float32) + b1[...]
        y1pad[:, _PAD + c0:_PAD + c0 + ch] = jnp.maximum(y1, 0.0)

    for ci in range(nch):
        c0 = ci * ch
        taps = [tap_y(kh - 1, kw - 1, c0)
                for kh in range(3) for kw in range(3)]
        d2 = jnp.concatenate(taps, axis=0)
        y2 = jnp.dot(w2f[...], d2, preferred_element_type=jnp.float32) + b2[...]
        res = jnp.dot(wr[...], pscr[0, :, c0:c0 + ch],
                      preferred_element_type=jnp.float32)
        o_ref[0, :, c0:c0 + ch] = jnp.maximum(y2 + res, 0.0)


def kernel(dw1, pw1, s1, b1, dw2, pw2, s2, b2, w1x1, x):
    n, cin, h, w = x.shape
    oh, ow = h // 2, w // 2
    hw = oh * ow
    m = hw
    cout = pw1.shape[1]

    xr = x.astype(jnp.bfloat16).reshape(n, cin, oh, 2, ow, 2)
    phases = [xr[:, :, :, a, :, b].reshape(n, cin, m)
              for a in (0, 1) for b in (0, 1)]

    w1f = ((dw1[:, :, None] * pw1[None, :, :]).reshape(9 * cin, cout).T
           * s1[:, None])
    w2f = ((dw2[:, :, None] * pw2[None, :, :]).reshape(9 * cout, cout).T
           * s2[:, None])
    wr = w1x1.T
    b1r = b1[:, None]
    b2r = b2[:, None]

    ch = min(hw, 2048)
    body = functools.partial(_block_kernel, cin=cin, ow=ow, m=m, ch=ch)

    resident = lambda s: pl.BlockSpec(s, lambda i: (0, 0))

    out = pl.pallas_call(
        body,
        out_shape=jax.ShapeDtypeStruct((n, cout, hw), jnp.float32),
        grid=(n,),
        in_specs=[pl.BlockSpec((1, cin, m), lambda i: (i, 0, 0))] * 4 + [
            resident(w1f.shape), resident(b1r.shape),
            resident(w2f.shape), resident(b2r.shape),
            resident(wr.shape),
        ],
        out_specs=pl.BlockSpec((1, cout, hw), lambda i: (i, 0, 0)),
        scratch_shapes=[pltpu.VMEM((cout, _PAD + m + 256), jnp.float32),
                        pltpu.VMEM((4, cin, m), jnp.float32)],
        compiler_params=pltpu.CompilerParams(
            dimension_semantics=("parallel",),
            vmem_limit_bytes=50 * 1024 * 1024,
        ),
    )(*phases, w1f, b1r, w2f, b2r, wr)

    return out.reshape(n, cout, oh, ow)
